# baseline (speedup 1.0000x reference)
import functools

import jax
import jax.numpy as jnp
from jax import lax
from jax.experimental import pallas as pl
from jax.experimental.pallas import tpu as pltpu
from jax.experimental.pallas import tpu_sc as plsc

NUM_CORES = 2
NUM_SUBCORES = 16
NW = NUM_CORES * NUM_SUBCORES
LANES = 16
TC = 2
BL = 128
OSTRIDE = TC * BL + 1


def _sc_encode(ridT, table, posT, *, B, T, N, D):
    n_ch = T // TC
    n_dg = D // LANES

    mesh = plsc.VectorSubcoreMesh(
        core_axis_name="c", subcore_axis_name="s",
        num_cores=NUM_CORES, num_subcores=NUM_SUBCORES)

    @functools.partial(
        pl.kernel,
        out_type=jax.ShapeDtypeStruct((D, T // 8, B // BL, 8, BL), jnp.float32),
        mesh=mesh,
        compiler_params=pltpu.CompilerParams(
            needs_layout_passes=False, use_tc_tiling_on_sc=False),
        scratch_types=[
            pltpu.VMEM((TC, BL), jnp.int32),
            pltpu.VMEM((TC, BL), jnp.int32),
            pltpu.VMEM((TC * BL, D), jnp.float32),
            pltpu.VMEM((TC * BL, D), jnp.float32),
            pltpu.VMEM((T, D), jnp.float32),
            pltpu.VMEM((D, OSTRIDE), jnp.float32),
            pltpu.VMEM((D, OSTRIDE), jnp.float32),
            pltpu.SemaphoreType.DMA,
            pltpu.SemaphoreType.DMA,
            pltpu.SemaphoreType.DMA,
            pltpu.SemaphoreType.DMA,
            pltpu.SemaphoreType.DMA,
            pltpu.SemaphoreType.DMA,
        ],
    )
    def sc_kernel(ridT_hbm, table_hbm, posT_hbm, out_hbm,
                  idx0, idx1, rows0, rows1, posT_v, outc0, outc1,
                  semi0, semi1, semg0, semg1, semo0, semo1):
        idx_v = (idx0, idx1)
        rows_v = (rows0, rows1)
        outc_v = (outc0, outc1)
        semi = (semi0, semi1)
        semg = (semg0, semg1)
        semo = (semo0, semo1)

        wid = lax.axis_index("s") * NUM_CORES + lax.axis_index("c")
        b0 = wid * BL
        pltpu.sync_copy(posT_hbm, posT_v)
        iota = lax.iota(jnp.int32, LANES)
        dvecs = [iota + dg * LANES for dg in range(n_dg)]

        def load_idx(buf, c):
            pltpu.async_copy(
                ridT_hbm.at[pl.ds(c * TC, TC), pl.ds(b0, BL)],
                idx_v[buf], semi[buf])

        def wait_idx(buf):
            pltpu.make_async_copy(
                ridT_hbm.at[pl.ds(0, TC), pl.ds(b0, BL)], idx_v[buf],
                semi[buf]).wait()

        def start_gather(buf):
            for tl in range(TC):
                pltpu.async_copy(
                    table_hbm.at[idx_v[buf].at[tl]],
                    rows_v[buf].at[pl.ds(tl * BL, BL)], semg[buf])

        def drain_gather(buf):
            pltpu.make_async_copy(
                table_hbm.at[pl.ds(0, TC * BL)], rows_v[buf],
                semg[buf]).wait()

        def out_dma(buf, c, fn, sem):
            t0 = c * TC
            tb = t0 // 8
            ts = t0 - tb * 8
            for tl in range(TC):
                fn(outc_v[buf].at[:, pl.ds(tl * BL, BL)],
                   out_hbm.at[:, tb, wid, ts + tl, :], sem)

        for buf in (0, 1):
            load_idx(buf, buf)
            wait_idx(buf)
            start_gather(buf)

        def body(j, carry):
            for buf in (0, 1):
                c = 2 * j + buf
                t0 = c * TC
                drain_gather(buf)
                load_idx(buf, jnp.minimum(c + 2, n_ch - 1))
                @pl.when(j > 0)
                def _():
                    for tl in range(TC):
                        pltpu.make_async_copy(
                            outc_v[buf].at[:, pl.ds(tl * BL, BL)],
                            out_hbm.at[:, 0, wid, tl, :],
                            semo[buf]).wait()

                pv = [[posT_v[t0 + tl, pl.ds(dg * LANES, LANES)]
                       for dg in range(n_dg)] for tl in range(TC)]

                @plsc.parallel_loop(0, BL, step=1, unroll=16)
                def blbody(bl):
                    for tl in range(TC):
                        col = jnp.full((LANES,), tl * BL + bl, jnp.int32)
                        row = tl * BL + bl
                        for dg in range(n_dg):
                            vec = (rows_v[buf][row, pl.ds(dg * LANES, LANES)]
                                   + pv[tl][dg])
                            plsc.store_scatter(
                                outc_v[buf], [dvecs[dg], col], vec)

                out_dma(buf, c, pltpu.async_copy, semo[buf])
                wait_idx(buf)
                start_gather(buf)
            return carry

        lax.fori_loop(0, n_ch // 2, body, 0)

        for buf in (0, 1):
            drain_gather(buf)
            for tl in range(TC):
                pltpu.make_async_copy(
                    outc_v[buf].at[:, pl.ds(tl * BL, BL)],
                    out_hbm.at[:, 0, wid, tl, :], semo[buf]).wait()

    return sc_kernel(ridT, table, posT)


def _tc_format(tableT, *, N, D):
    NB = 32768

    def body(in_ref, out_ref):
        for j in range(NB // 256):
            x = in_ref[:, pl.ds(j * 256, 256)]
            out_ref[pl.ds(j * 256, 256), :D] = jnp.transpose(x)

    return pl.pallas_call(
        body,
        grid=(pl.cdiv(N, NB),),
        in_specs=[pl.BlockSpec((D, NB), lambda i: (0, i))],
        out_specs=pl.BlockSpec((NB, 128), lambda i: (i, 0)),
        out_shape=jax.ShapeDtypeStruct((N, 128), jnp.float32),
    )(tableT)


def kernel(rid, table, pos):
    B, T = rid.shape
    N, D = table.shape
    tableT = jnp.transpose(table)
    tableL = _tc_format(tableT, N=N, D=D).reshape(2 * N, D)
    ridT = jnp.transpose(rid.astype(jnp.int32)) * 2
    posT = jnp.transpose(pos[0].astype(jnp.float32))
    out5 = _sc_encode(ridT, tableL, posT, B=B, T=T, N=2 * N, D=D)
    x = jnp.transpose(out5, (0, 1, 3, 2, 4))
    x = jnp.reshape(x, (D, T, B))
    return jnp.transpose(x, (2, 0, 1))


# speedup vs baseline: 1.0014x; 1.0014x over previous
import functools

import jax
import jax.numpy as jnp
from jax import lax
from jax.experimental import pallas as pl
from jax.experimental.pallas import tpu as pltpu
from jax.experimental.pallas import tpu_sc as plsc

NUM_CORES = 2
NUM_SUBCORES = 16
NW = NUM_CORES * NUM_SUBCORES
LANES = 16
TC = 2
BL = 128
OSTRIDE = TC * BL + 1


def _sc_encode(ridT, table, posT, *, B, T, N, D):
    n_ch = T // TC
    n_dg = D // LANES

    mesh = plsc.VectorSubcoreMesh(
        core_axis_name="c", subcore_axis_name="s",
        num_cores=NUM_CORES, num_subcores=NUM_SUBCORES)

    @functools.partial(
        pl.kernel,
        out_type=jax.ShapeDtypeStruct((D, T // 8, B // BL, 8, BL), jnp.float32),
        mesh=mesh,
        compiler_params=pltpu.CompilerParams(
            needs_layout_passes=False, use_tc_tiling_on_sc=False),
        scratch_types=[
            pltpu.VMEM((TC, BL), jnp.int32),
            pltpu.VMEM((TC, BL), jnp.int32),
            pltpu.VMEM((TC * BL, D), jnp.float32),
            pltpu.VMEM((TC * BL, D), jnp.float32),
            pltpu.VMEM((T, D), jnp.float32),
            pltpu.VMEM((D, OSTRIDE), jnp.float32),
            pltpu.VMEM((D, OSTRIDE), jnp.float32),
            pltpu.SemaphoreType.DMA,
            pltpu.SemaphoreType.DMA,
            pltpu.SemaphoreType.DMA,
            pltpu.SemaphoreType.DMA,
            pltpu.SemaphoreType.DMA,
            pltpu.SemaphoreType.DMA,
        ],
    )
    def sc_kernel(ridT_hbm, table_hbm, posT_hbm, out_hbm,
                  idx0, idx1, rows0, rows1, posT_v, outc0, outc1,
                  semi0, semi1, semg0, semg1, semo0, semo1):
        idx_v = (idx0, idx1)
        rows_v = (rows0, rows1)
        outc_v = (outc0, outc1)
        semi = (semi0, semi1)
        semg = (semg0, semg1)
        semo = (semo0, semo1)

        wid = lax.axis_index("s") * NUM_CORES + lax.axis_index("c")
        b0 = wid * BL
        pltpu.sync_copy(posT_hbm, posT_v)
        iota = lax.iota(jnp.int32, LANES)
        dvecs = [iota + dg * LANES for dg in range(n_dg)]

        def load_idx(buf, c):
            pltpu.async_copy(
                ridT_hbm.at[pl.ds(c * TC, TC), pl.ds(b0, BL)],
                idx_v[buf], semi[buf])

        def wait_idx(buf):
            pltpu.make_async_copy(
                ridT_hbm.at[pl.ds(0, TC), pl.ds(b0, BL)], idx_v[buf],
                semi[buf]).wait()

        def start_gather(buf):
            for tl in range(TC):
                pltpu.async_copy(
                    table_hbm.at[idx_v[buf].at[tl]],
                    rows_v[buf].at[pl.ds(tl * BL, BL)], semg[buf])

        def drain_gather(buf):
            pltpu.make_async_copy(
                table_hbm.at[pl.ds(0, TC * BL)], rows_v[buf],
                semg[buf]).wait()

        def out_dma(buf, c, fn, sem):
            t0 = c * TC
            tb = t0 // 8
            ts = t0 - tb * 8
            for tl in range(TC):
                fn(outc_v[buf].at[:, pl.ds(tl * BL, BL)],
                   out_hbm.at[:, tb, wid, ts + tl, :], sem)

        for buf in (0, 1):
            load_idx(buf, buf)
            wait_idx(buf)
            start_gather(buf)

        def body(j, carry):
            for buf in (0, 1):
                c = 2 * j + buf
                t0 = c * TC
                drain_gather(buf)
                load_idx(buf, jnp.minimum(c + 2, n_ch - 1))
                @pl.when(j > 0)
                def _():
                    for tl in range(TC):
                        pltpu.make_async_copy(
                            outc_v[buf].at[:, pl.ds(tl * BL, BL)],
                            out_hbm.at[:, 0, wid, tl, :],
                            semo[buf]).wait()

                pv = [[posT_v[t0 + tl, pl.ds(dg * LANES, LANES)]
                       for dg in range(n_dg)] for tl in range(TC)]

                @plsc.parallel_loop(0, BL, step=1, unroll=8)
                def blbody(bl):
                    for tl in range(TC):
                        col = jnp.full((LANES,), tl * BL + bl, jnp.int32)
                        row = tl * BL + bl
                        for dg in range(n_dg):
                            vec = (rows_v[buf][row, pl.ds(dg * LANES, LANES)]
                                   + pv[tl][dg])
                            plsc.store_scatter(
                                outc_v[buf], [dvecs[dg], col], vec)

                out_dma(buf, c, pltpu.async_copy, semo[buf])
                wait_idx(buf)
                start_gather(buf)
            return carry

        lax.fori_loop(0, n_ch // 2, body, 0)

        for buf in (0, 1):
            drain_gather(buf)
            for tl in range(TC):
                pltpu.make_async_copy(
                    outc_v[buf].at[:, pl.ds(tl * BL, BL)],
                    out_hbm.at[:, 0, wid, tl, :], semo[buf]).wait()

    return sc_kernel(ridT, table, posT)


def _tc_format(tableT, *, N, D):
    NB = 32768

    def body(in_ref, out_ref):
        for j in range(NB // 256):
            x = in_ref[:, pl.ds(j * 256, 256)]
            out_ref[pl.ds(j * 256, 256), :D] = jnp.transpose(x)

    return pl.pallas_call(
        body,
        grid=(pl.cdiv(N, NB),),
        in_specs=[pl.BlockSpec((D, NB), lambda i: (0, i))],
        out_specs=pl.BlockSpec((NB, 128), lambda i: (i, 0)),
        out_shape=jax.ShapeDtypeStruct((N, 128), jnp.float32),
    )(tableT)


def kernel(rid, table, pos):
    B, T = rid.shape
    N, D = table.shape
    tableT = jnp.transpose(table)
    tableL = _tc_format(tableT, N=N, D=D).reshape(2 * N, D)
    ridT = jnp.transpose(rid.astype(jnp.int32)) * 2
    posT = jnp.transpose(pos[0].astype(jnp.float32))
    out5 = _sc_encode(ridT, tableL, posT, B=B, T=T, N=2 * N, D=D)
    x = jnp.transpose(out5, (0, 1, 3, 2, 4))
    x = jnp.reshape(x, (D, T, B))
    return jnp.transpose(x, (2, 0, 1))
